# Initial kernel scaffold; baseline (speedup 1.0000x reference)
import jax
import jax.numpy as jnp
from jax.experimental import pallas as pl


def kernel(x_, edge_index, params, noise):
    raise NotImplementedError("write your pallas kernel here")


# speedup vs baseline: 1.0038x; 1.0038x over previous
import functools

import jax
import jax.numpy as jnp
from jax import lax
from jax.experimental import pallas as pl
from jax.experimental.pallas import tpu as pltpu

_N = 10000
_D = 128
_H = 8
_HD = 16
_HID = 1024

_BA = 1000
_BD = 400
_BE = 1000


def _proj_body(xc_ref, wc_ref, bc_ref, wq_ref, bq_ref, wk_ref, bk_ref,
               wv_ref, bv_ref, sel_ref,
               v_ref, qk_ref, qh_ref, kh_ref, q2h_ref, k2h_ref, vh_ref,
               v2h_ref):
    xc = xc_ref[...]
    x = jnp.dot(xc, wc_ref[...], preferred_element_type=jnp.float32) + bc_ref[...]
    q = jnp.dot(x, wq_ref[...], preferred_element_type=jnp.float32) + bq_ref[...]
    k = jnp.dot(x, wk_ref[...], preferred_element_type=jnp.float32) + bk_ref[...]
    v = jnp.dot(x, wv_ref[...], preferred_element_type=jnp.float32) + bv_ref[...]
    v_ref[...] = v
    sel = sel_ref[...]
    dotf32 = functools.partial(jnp.dot, preferred_element_type=jnp.float32, precision=lax.Precision.HIGHEST)
    qk_ref[...] = dotf32(q * k, sel)
    qh_ref[...] = dotf32(q, sel)
    kh_ref[...] = dotf32(k, sel)
    q2h_ref[...] = dotf32(q * q, sel)
    k2h_ref[...] = dotf32(k * k, sel)
    vh_ref[...] = dotf32(v, sel)
    v2h_ref[...] = dotf32(v * v, sel)


def _proj_call(x_concat, params, sel):
    nb = _N // _BA
    h8 = jax.ShapeDtypeStruct((_N, _H), jnp.float32)
    row = lambda i: (i, 0)
    const = lambda i: (0, 0)
    return pl.pallas_call(
        _proj_body,
        grid=(nb,),
        in_specs=[
            pl.BlockSpec((_BA, 5 * _D), row),
            pl.BlockSpec((5 * _D, _D), const),
            pl.BlockSpec((1, _D), const),
            pl.BlockSpec((_D, _D), const),
            pl.BlockSpec((1, _D), const),
            pl.BlockSpec((_D, _D), const),
            pl.BlockSpec((1, _D), const),
            pl.BlockSpec((_D, _D), const),
            pl.BlockSpec((1, _D), const),
            pl.BlockSpec((_D, _H), const),
        ],
        out_specs=[
            pl.BlockSpec((_BA, _D), row),
            pl.BlockSpec((_BA, _H), row),
            pl.BlockSpec((_BA, _H), row),
            pl.BlockSpec((_BA, _H), row),
            pl.BlockSpec((_BA, _H), row),
            pl.BlockSpec((_BA, _H), row),
            pl.BlockSpec((_BA, _H), row),
            pl.BlockSpec((_BA, _H), row),
        ],
        out_shape=[
            jax.ShapeDtypeStruct((_N, _D), jnp.float32),
            h8, h8, h8, h8, h8, h8, h8,
        ],
    )(x_concat, params['Wc'], params['bc'].reshape(1, _D),
      params['Wq'], params['bq'].reshape(1, _D),
      params['Wk'], params['bk'].reshape(1, _D),
      params['Wv'], params['bv'].reshape(1, _D), sel)


def _sim_body(qk_ref, qh_ref, kh_ref, q2h_ref, k2h_ref, vh_ref, v2h_ref,
              noise_ref, sim_ref, sc_ref):
    m = float(_N * _D)
    qh = qh_ref[...]
    kh = kh_ref[...]
    sq = jnp.sum(qh)
    sk = jnp.sum(kh)
    ssq = jnp.sum(q2h_ref[...])
    ssk = jnp.sum(k2h_ref[...])
    sv = jnp.sum(vh_ref[...])
    ssv = jnp.sum(v2h_ref[...])
    qm = sq / m
    km = sk / m
    qs = jnp.sqrt((ssq - sq * sq / m) / (m - 1.0))
    ks = jnp.sqrt((ssk - sk * sk / m) / (m - 1.0))
    vs = jnp.sqrt((ssv - sv * sv / m) / (m - 1.0))
    score = (qk_ref[...] - km * qh - qm * kh + (_HD * qm * km)) / (qs * ks)
    score = score - jnp.max(score, axis=1, keepdims=True)
    p = jnp.exp(score)
    p = p / jnp.sum(p, axis=1, keepdims=True)
    ms = float(_N * _H)
    sp = jnp.sum(p)
    ssp = jnp.sum(p * p)
    pm = sp / ms
    ps = jnp.sqrt((ssp - sp * sp / ms) / (ms - 1.0))
    sim_ref[...] = (p - pm) / ps + noise_ref[...] * 0.004
    lane = lax.broadcasted_iota(jnp.int32, (1, _D), 1)
    vm = qm
    sc_ref[...] = jnp.where(lane == 0, vm,
                            jnp.where(lane == 1, 1.0 / vs,
                                      jnp.zeros((1, _D), jnp.float32)))


def _sim_call(qk, qh, kh, q2h, k2h, vh, v2h, noise):
    full = pl.BlockSpec((_N, _H), lambda: (0, 0))
    return pl.pallas_call(
        _sim_body,
        grid=(),
        in_specs=[full] * 8,
        out_specs=[full, pl.BlockSpec((1, _D), lambda: (0, 0))],
        out_shape=[jax.ShapeDtypeStruct((_N, _H), jnp.float32),
                   jax.ShapeDtypeStruct((1, _D), jnp.float32)],
    )(qk, qh, kh, q2h, k2h, vh, v2h, noise)


def _flash_body(simb_ref, sim_ref, v_ref, sc_ref, out_ref):
    i = pl.program_id(0)
    sim_i = simb_ref[...]
    sim_all = sim_ref[...]
    a = lax.dot_general(sim_i, sim_all, (((1,), (1,)), ((), ())),
                        preferred_element_type=jnp.float32)
    mx = jnp.max(a, axis=1, keepdims=True)
    p = jnp.exp(a - mx)
    r = jnp.sum(p, axis=1, keepdims=True)
    s = p / r
    vm = sc_ref[0, 0]
    ivs = sc_ref[0, 1]
    w = (v_ref[...] - vm) * ivs
    contrib = lax.dot_general(s, w, (((0,), (0,)), ((), ())),
                              preferred_element_type=jnp.float32)

    @pl.when(i == 0)
    def _():
        out_ref[...] = jnp.zeros_like(out_ref)

    out_ref[...] += contrib


def _flash_call(sim, v, sc):
    nb = _N // _BD
    return pl.pallas_call(
        _flash_body,
        grid=(nb,),
        in_specs=[
            pl.BlockSpec((_BD, _H), lambda i: (i, 0)),
            pl.BlockSpec((_N, _H), lambda i: (0, 0)),
            pl.BlockSpec((_BD, _D), lambda i: (i, 0)),
            pl.BlockSpec(memory_space=pltpu.SMEM),
        ],
        out_specs=pl.BlockSpec((_N, _D), lambda i: (0, 0)),
        out_shape=jax.ShapeDtypeStruct((_N, _D), jnp.float32),
    )(sim, sim, v, sc)


def _ffn_body(y_ref, w1_ref, b1_ref, w2_ref, b2_ref, o_ref):
    h = jnp.dot(y_ref[...], w1_ref[...],
                preferred_element_type=jnp.float32) + b1_ref[...]
    h = 0.5 * h * (1.0 + lax.erf(h * 0.7071067811865476))
    o_ref[...] = jnp.dot(h, w2_ref[...],
                         preferred_element_type=jnp.float32) + b2_ref[...]


def _ffn_call(y, params):
    nb = _N // _BE
    const = lambda i: (0, 0)
    return pl.pallas_call(
        _ffn_body,
        grid=(nb,),
        in_specs=[
            pl.BlockSpec((_BE, _D), lambda i: (i, 0)),
            pl.BlockSpec((_D, _HID), const),
            pl.BlockSpec((1, _HID), const),
            pl.BlockSpec((_HID, _D), const),
            pl.BlockSpec((1, _D), const),
        ],
        out_specs=pl.BlockSpec((_BE, _D), lambda i: (i, 0)),
        out_shape=jax.ShapeDtypeStruct((_N, _D), jnp.float32),
    )(y, params['W1'], params['b1'].reshape(1, _HID),
      params['W2'], params['b2'].reshape(1, _D))


def _edge_stage(x_, edge_index):
    src = edge_index[0]
    dst = edge_index[1]
    deg = jnp.bincount(src, length=_N).astype(jnp.float32)
    prod = deg[src] * deg[dst]
    norm = 1.0 / jnp.sqrt(prod)
    indeg = jnp.bincount(dst, length=_N).astype(jnp.float32)
    denom = jnp.maximum(indeg, 1.0)[:, None]
    xs = x_[src]
    nb = norm[:, None]

    def seg_mean(mat):
        return jax.ops.segment_sum(mat, dst, num_segments=_N) / denom

    sn = jax.ops.segment_sum(norm, dst, num_segments=_N)[:, None]
    sx = jax.ops.segment_sum(xs, dst, num_segments=_N)
    x_0 = (sx + sn) / denom
    x_1 = (sx - sn) / denom
    x_2 = seg_mean(xs * nb)
    x_3 = seg_mean(xs / nb)
    return jnp.concatenate([x_1, x_2, x_0, x_, x_3], axis=1)


def kernel(x_, edge_index, params, noise):
    x_concat = _edge_stage(x_, edge_index)
    sel = jnp.repeat(jnp.eye(_H, dtype=jnp.float32), _HD, axis=0)
    v, qk, qh, kh, q2h, k2h, vh, v2h = _proj_call(x_concat, params, sel)
    sim, sc = _sim_call(qk, qh, kh, q2h, k2h, vh, v2h, noise)
    y = _flash_call(sim, v, sc)
    return _ffn_call(y, params)


# speedup vs baseline: 3.8032x; 3.7889x over previous
import functools

import jax
import jax.numpy as jnp
from jax import lax
from jax.experimental import pallas as pl
from jax.experimental.pallas import tpu as pltpu

_N = 10000
_D = 128
_H = 8
_HD = 16
_HID = 1024

_BA = 1000
_BD = 400
_BE = 1000


def _proj_body(xc_ref, wc_ref, bc_ref, wq_ref, bq_ref, wk_ref, bk_ref,
               wv_ref, bv_ref, sel_ref,
               v_ref, qk_ref, qh_ref, kh_ref, q2h_ref, k2h_ref, vh_ref,
               v2h_ref):
    xc = xc_ref[...]
    x = jnp.dot(xc, wc_ref[...], preferred_element_type=jnp.float32) + bc_ref[...]
    q = jnp.dot(x, wq_ref[...], preferred_element_type=jnp.float32) + bq_ref[...]
    k = jnp.dot(x, wk_ref[...], preferred_element_type=jnp.float32) + bk_ref[...]
    v = jnp.dot(x, wv_ref[...], preferred_element_type=jnp.float32) + bv_ref[...]
    v_ref[...] = v
    sel = sel_ref[...]
    dotf32 = functools.partial(jnp.dot, preferred_element_type=jnp.float32, precision=lax.Precision.HIGHEST)
    qk_ref[...] = dotf32(q * k, sel)
    qh_ref[...] = dotf32(q, sel)
    kh_ref[...] = dotf32(k, sel)
    q2h_ref[...] = dotf32(q * q, sel)
    k2h_ref[...] = dotf32(k * k, sel)
    vh_ref[...] = dotf32(v, sel)
    v2h_ref[...] = dotf32(v * v, sel)


def _proj_call(x_concat, params, sel):
    nb = _N // _BA
    h8 = jax.ShapeDtypeStruct((_N, _H), jnp.float32)
    row = lambda i: (i, 0)
    const = lambda i: (0, 0)
    return pl.pallas_call(
        _proj_body,
        grid=(nb,),
        in_specs=[
            pl.BlockSpec((_BA, 5 * _D), row),
            pl.BlockSpec((5 * _D, _D), const),
            pl.BlockSpec((1, _D), const),
            pl.BlockSpec((_D, _D), const),
            pl.BlockSpec((1, _D), const),
            pl.BlockSpec((_D, _D), const),
            pl.BlockSpec((1, _D), const),
            pl.BlockSpec((_D, _D), const),
            pl.BlockSpec((1, _D), const),
            pl.BlockSpec((_D, _H), const),
        ],
        out_specs=[
            pl.BlockSpec((_BA, _D), row),
            pl.BlockSpec((_BA, _H), row),
            pl.BlockSpec((_BA, _H), row),
            pl.BlockSpec((_BA, _H), row),
            pl.BlockSpec((_BA, _H), row),
            pl.BlockSpec((_BA, _H), row),
            pl.BlockSpec((_BA, _H), row),
            pl.BlockSpec((_BA, _H), row),
        ],
        out_shape=[
            jax.ShapeDtypeStruct((_N, _D), jnp.float32),
            h8, h8, h8, h8, h8, h8, h8,
        ],
    )(x_concat, params['Wc'], params['bc'].reshape(1, _D),
      params['Wq'], params['bq'].reshape(1, _D),
      params['Wk'], params['bk'].reshape(1, _D),
      params['Wv'], params['bv'].reshape(1, _D), sel)


def _sim_body(qk_ref, qh_ref, kh_ref, q2h_ref, k2h_ref, vh_ref, v2h_ref,
              noise_ref, sim_ref, sc_ref):
    m = float(_N * _D)
    qh = qh_ref[...]
    kh = kh_ref[...]
    sq = jnp.sum(qh)
    sk = jnp.sum(kh)
    ssq = jnp.sum(q2h_ref[...])
    ssk = jnp.sum(k2h_ref[...])
    sv = jnp.sum(vh_ref[...])
    ssv = jnp.sum(v2h_ref[...])
    qm = sq / m
    km = sk / m
    qs = jnp.sqrt((ssq - sq * sq / m) / (m - 1.0))
    ks = jnp.sqrt((ssk - sk * sk / m) / (m - 1.0))
    vs = jnp.sqrt((ssv - sv * sv / m) / (m - 1.0))
    score = (qk_ref[...] - km * qh - qm * kh + (_HD * qm * km)) / (qs * ks)
    score = score - jnp.max(score, axis=1, keepdims=True)
    p = jnp.exp(score)
    p = p / jnp.sum(p, axis=1, keepdims=True)
    ms = float(_N * _H)
    sp = jnp.sum(p)
    ssp = jnp.sum(p * p)
    pm = sp / ms
    ps = jnp.sqrt((ssp - sp * sp / ms) / (ms - 1.0))
    sim_ref[...] = (p - pm) / ps + noise_ref[...] * 0.004
    lane = lax.broadcasted_iota(jnp.int32, (1, _D), 1)
    vm = qm
    sc_ref[...] = jnp.where(lane == 0, vm,
                            jnp.where(lane == 1, 1.0 / vs,
                                      jnp.zeros((1, _D), jnp.float32)))


def _sim_call(qk, qh, kh, q2h, k2h, vh, v2h, noise):
    full = pl.BlockSpec((_N, _H), lambda: (0, 0))
    return pl.pallas_call(
        _sim_body,
        grid=(),
        in_specs=[full] * 8,
        out_specs=[full, pl.BlockSpec((1, _D), lambda: (0, 0))],
        out_shape=[jax.ShapeDtypeStruct((_N, _H), jnp.float32),
                   jax.ShapeDtypeStruct((1, _D), jnp.float32)],
    )(qk, qh, kh, q2h, k2h, vh, v2h, noise)


def _flash_body(simb_ref, sim_ref, v_ref, sc_ref, out_ref):
    i = pl.program_id(0)
    sim_i = simb_ref[...]
    sim_all = sim_ref[...]
    a = lax.dot_general(sim_i, sim_all, (((1,), (1,)), ((), ())),
                        preferred_element_type=jnp.float32)
    mx = jnp.max(a, axis=1, keepdims=True)
    p = jnp.exp(a - mx)
    r = jnp.sum(p, axis=1, keepdims=True)
    s = p / r
    vm = sc_ref[0, 0]
    ivs = sc_ref[0, 1]
    w = (v_ref[...] - vm) * ivs
    contrib = lax.dot_general(s, w, (((0,), (0,)), ((), ())),
                              preferred_element_type=jnp.float32)

    @pl.when(i == 0)
    def _():
        out_ref[...] = jnp.zeros_like(out_ref)

    out_ref[...] += contrib


def _flash_call(sim, v, sc):
    nb = _N // _BD
    return pl.pallas_call(
        _flash_body,
        grid=(nb,),
        in_specs=[
            pl.BlockSpec((_BD, _H), lambda i: (i, 0)),
            pl.BlockSpec((_N, _H), lambda i: (0, 0)),
            pl.BlockSpec((_BD, _D), lambda i: (i, 0)),
            pl.BlockSpec(memory_space=pltpu.SMEM),
        ],
        out_specs=pl.BlockSpec((_N, _D), lambda i: (0, 0)),
        out_shape=jax.ShapeDtypeStruct((_N, _D), jnp.float32),
    )(sim, sim, v, sc)


def _ffn_body(y_ref, w1_ref, b1_ref, w2_ref, b2_ref, o_ref):
    h = jnp.dot(y_ref[...], w1_ref[...],
                preferred_element_type=jnp.float32) + b1_ref[...]
    h = 0.5 * h * (1.0 + lax.erf(h * 0.7071067811865476))
    o_ref[...] = jnp.dot(h, w2_ref[...],
                         preferred_element_type=jnp.float32) + b2_ref[...]


def _ffn_call(y, params):
    nb = _N // _BE
    const = lambda i: (0, 0)
    return pl.pallas_call(
        _ffn_body,
        grid=(nb,),
        in_specs=[
            pl.BlockSpec((_BE, _D), lambda i: (i, 0)),
            pl.BlockSpec((_D, _HID), const),
            pl.BlockSpec((1, _HID), const),
            pl.BlockSpec((_HID, _D), const),
            pl.BlockSpec((1, _D), const),
        ],
        out_specs=pl.BlockSpec((_BE, _D), lambda i: (i, 0)),
        out_shape=jax.ShapeDtypeStruct((_N, _D), jnp.float32),
    )(y, params['W1'], params['b1'].reshape(1, _HID),
      params['W2'], params['b2'].reshape(1, _D))


_BW = 400
_YC = 512


def _spmm_body(w_ref, y_ref, z_ref):
    z_ref[...] = jnp.dot(w_ref[...], y_ref[...],
                         preferred_element_type=jnp.float32,
                         precision=lax.Precision.HIGHEST)


def _spmm_call(w, y):
    nb = _N // _BW
    return pl.pallas_call(
        _spmm_body,
        grid=(nb,),
        in_specs=[
            pl.BlockSpec((_BW, _N), lambda i: (i, 0)),
            pl.BlockSpec((_N, _YC), lambda i: (0, 0)),
        ],
        out_specs=pl.BlockSpec((_BW, _YC), lambda i: (i, 0)),
        out_shape=jax.ShapeDtypeStruct((_N, _YC), jnp.float32),
    )(w, y)


def _edge_stage(x_, edge_index):
    src = edge_index[0]
    dst = edge_index[1]
    deg = jnp.bincount(src, length=_N).astype(jnp.float32)
    indeg = jnp.bincount(dst, length=_N).astype(jnp.float32)
    denom = jnp.maximum(indeg, 1.0)[:, None]
    a = jnp.where(deg > 0, lax.rsqrt(jnp.maximum(deg, 1.0)), 0.0)
    b = jnp.sqrt(deg)
    w = jnp.zeros((_N, _N), jnp.float32).at[dst, src].add(1.0)
    y = jnp.concatenate(
        [x_, x_ * a[:, None], x_ * b[:, None], a[:, None],
         jnp.zeros((_N, _YC - 3 * _D - 1), jnp.float32)], axis=1)
    z = _spmm_call(w, y)
    sx = z[:, :_D]
    sn = (a * z[:, 3 * _D])[:, None]
    x_0 = (sx + sn) / denom
    x_1 = (sx - sn) / denom
    x_2 = a[:, None] * z[:, _D:2 * _D] / denom
    x_3 = b[:, None] * z[:, 2 * _D:3 * _D] / denom
    return jnp.concatenate([x_1, x_2, x_0, x_, x_3], axis=1)


def kernel(x_, edge_index, params, noise):
    x_concat = _edge_stage(x_, edge_index)
    sel = jnp.repeat(jnp.eye(_H, dtype=jnp.float32), _HD, axis=0)
    v, qk, qh, kh, q2h, k2h, vh, v2h = _proj_call(x_concat, params, sel)
    sim, sc = _sim_call(qk, qh, kh, q2h, k2h, vh, v2h, noise)
    y = _flash_call(sim, v, sc)
    return _ffn_call(y, params)
